# baseline (speedup 1.0000x reference)
import functools

import jax
import jax.numpy as jnp
from jax import lax
from jax.experimental import pallas as pl
from jax.experimental.pallas import tpu as pltpu
from jax.experimental.pallas import tpu_sc as plsc

MEMORY_SIZE = 512
CONTRAST_SIZE = 64

_NUM_CORES = 2
_NUM_SUBCORES = 16
_NW = _NUM_CORES * _NUM_SUBCORES

_B = 16
_C = 128
_P = 32 * 32
_FROWS = _B * _C
_RPW = _FROWS // _NW
_NBUF = 2
_RCH = _RPW // _NBUF


@functools.partial(
    pl.kernel,
    mesh=plsc.VectorSubcoreMesh(core_axis_name="c", subcore_axis_name="s"),
    out_type=[
        jax.ShapeDtypeStruct((_FROWS, _P), jnp.float32),
        jax.ShapeDtypeStruct((_B, _P), jnp.float32),
    ],
    scratch_types=[
        pltpu.VMEM((_NBUF, _RCH, _P), jnp.float32),
        pltpu.VMEM((1, _P), jnp.float32),
        pltpu.SemaphoreType.DMA,
        pltpu.SemaphoreType.DMA,
        pltpu.SemaphoreType.DMA,
    ],
    compiler_params=pltpu.CompilerParams(use_tc_tiling_on_sc=True),
)
def _routed_gather(feat_hbm, lab_hbm, out_f_hbm, out_l_hbm,
                   fbuf, lbuf, rsem, wsem, lsem):
    wid = lax.axis_index("s") * _NUM_CORES + lax.axis_index("c")
    rbase = wid * _RPW

    @pl.when(wid < _B)
    def _():
        pltpu.make_async_copy(lab_hbm.at[pl.ds(wid, 1)], lbuf, lsem).start()

    reads = []
    for b in range(_NBUF):
        rd = pltpu.make_async_copy(
            feat_hbm.at[pl.ds(rbase + b * _RCH, _RCH)], fbuf.at[b], rsem)
        rd.start()
        reads.append(rd)
    writes = []
    for b in range(_NBUF):
        reads[b].wait()
        wr = pltpu.make_async_copy(
            fbuf.at[b], out_f_hbm.at[pl.ds(rbase + b * _RCH, _RCH)], wsem)
        wr.start()
        writes.append(wr)

    @pl.when(wid < _B)
    def _():
        pltpu.make_async_copy(lab_hbm.at[pl.ds(wid, 1)], lbuf, lsem).wait()
        wr = pltpu.make_async_copy(lbuf, out_l_hbm.at[pl.ds(wid, 1)], lsem)
        wr.start()
        wr.wait()

    for wr in writes:
        wr.wait()


def kernel(t_feats, labels, teacher_feature_queue, teacher_mask_queue):
    del teacher_feature_queue, teacher_mask_queue
    keys = jax.lax.stop_gradient(t_feats)
    labs = jax.lax.stop_gradient(labels.astype(jnp.float32))
    out_f, out_l = _routed_gather(
        keys.reshape(_FROWS, _P), labs.reshape(_B, _P))
    return (
        out_f.reshape(t_feats.shape),
        out_l.reshape(labels.shape),
    )


# speedup vs baseline: 3.1805x; 3.1805x over previous
import functools

import jax
import jax.numpy as jnp
from jax import lax
from jax.experimental import pallas as pl
from jax.experimental.pallas import tpu as pltpu
from jax.experimental.pallas import tpu_sc as plsc

MEMORY_SIZE = 512
CONTRAST_SIZE = 64

_NUM_CORES = 2
_NUM_SUBCORES = 16
_NW = _NUM_CORES * _NUM_SUBCORES

_B = 16
_C = 128
_H = 32
_W = 32
_FROWS = _B * _H * _W
_RPW = _FROWS // _NW
_NBUF = 2
_RCH = _RPW // _NBUF


@functools.partial(
    pl.kernel,
    mesh=plsc.VectorSubcoreMesh(core_axis_name="c", subcore_axis_name="s"),
    out_type=[
        jax.ShapeDtypeStruct((_FROWS, _C), jnp.float32),
        jax.ShapeDtypeStruct((_B, 1, _H, _W), jnp.float32),
    ],
    scratch_types=[
        pltpu.VMEM((_NBUF, _RCH, _C), jnp.float32),
        pltpu.VMEM((1, _H, _W), jnp.float32),
        pltpu.SemaphoreType.DMA,
        pltpu.SemaphoreType.DMA,
        pltpu.SemaphoreType.DMA,
    ],
    compiler_params=pltpu.CompilerParams(use_tc_tiling_on_sc=True),
)
def _routed_gather(feat_hbm, lab_hbm, out_f_hbm, out_l_hbm,
                   fbuf, lbuf, rsem, wsem, lsem):
    wid = lax.axis_index("s") * _NUM_CORES + lax.axis_index("c")
    rbase = wid * _RPW

    @pl.when(wid < _B)
    def _():
        pltpu.make_async_copy(lab_hbm.at[wid], lbuf, lsem).start()

    reads = []
    for b in range(_NBUF):
        rd = pltpu.make_async_copy(
            feat_hbm.at[pl.ds(rbase + b * _RCH, _RCH)], fbuf.at[b], rsem)
        rd.start()
        reads.append(rd)
    writes = []
    for b in range(_NBUF):
        reads[b].wait()
        wr = pltpu.make_async_copy(
            fbuf.at[b], out_f_hbm.at[pl.ds(rbase + b * _RCH, _RCH)], wsem)
        wr.start()
        writes.append(wr)

    @pl.when(wid < _B)
    def _():
        pltpu.make_async_copy(lab_hbm.at[wid], lbuf, lsem).wait()
        wr = pltpu.make_async_copy(lbuf, out_l_hbm.at[wid], lsem)
        wr.start()
        wr.wait()

    for wr in writes:
        wr.wait()


def kernel(t_feats, labels, teacher_feature_queue, teacher_mask_queue):
    del teacher_feature_queue, teacher_mask_queue
    keys = jax.lax.stop_gradient(t_feats)
    labs = jax.lax.stop_gradient(labels.astype(jnp.float32))
    feat2d = keys.transpose(0, 2, 3, 1).reshape(_FROWS, _C)
    out_f, out_l = _routed_gather(feat2d, labs)
    out_f = out_f.reshape(_B, _H, _W, _C).transpose(0, 3, 1, 2)
    return (out_f, out_l)


# speedup vs baseline: 3.3487x; 1.0529x over previous
import functools

import jax
import jax.numpy as jnp
from jax import lax
from jax.experimental import pallas as pl
from jax.experimental.pallas import tpu as pltpu
from jax.experimental.pallas import tpu_sc as plsc

MEMORY_SIZE = 512
CONTRAST_SIZE = 64

_NUM_CORES = 2
_NUM_SUBCORES = 16
_NW = _NUM_CORES * _NUM_SUBCORES

_B = 16
_C = 128
_H = 32
_W = 32
_FROWS = _B * _H * _W
_BLK = 2048


@functools.partial(
    pl.kernel,
    mesh=plsc.VectorSubcoreMesh(core_axis_name="c", subcore_axis_name="s"),
    out_type=jax.ShapeDtypeStruct((_B, 1, _H, _W), jnp.float32),
    scratch_types=[
        pltpu.VMEM((1, _H, _W), jnp.float32),
        pltpu.SemaphoreType.DMA,
    ],
    compiler_params=pltpu.CompilerParams(use_tc_tiling_on_sc=True),
)
def _mask_gather(lab_hbm, out_l_hbm, lbuf, lsem):
    wid = lax.axis_index("s") * _NUM_CORES + lax.axis_index("c")

    @pl.when(wid < _B)
    def _():
        rd = pltpu.make_async_copy(lab_hbm.at[wid], lbuf, lsem)
        rd.start()
        rd.wait()
        wr = pltpu.make_async_copy(lbuf, out_l_hbm.at[wid], lsem)
        wr.start()
        wr.wait()


def _feat_body(src_ref, dst_ref):
    dst_ref[...] = src_ref[...]


_feat_stream = pl.pallas_call(
    _feat_body,
    grid=(_FROWS // _BLK,),
    in_specs=[pl.BlockSpec((_BLK, _C), lambda i: (i, 0))],
    out_specs=pl.BlockSpec((_BLK, _C), lambda i: (i, 0)),
    out_shape=jax.ShapeDtypeStruct((_FROWS, _C), jnp.float32),
)


def kernel(t_feats, labels, teacher_feature_queue, teacher_mask_queue):
    del teacher_feature_queue, teacher_mask_queue
    keys = jax.lax.stop_gradient(t_feats)
    labs = jax.lax.stop_gradient(labels.astype(jnp.float32))
    feat2d = keys.transpose(0, 2, 3, 1).reshape(_FROWS, _C)
    out_l = _mask_gather(labs)
    out_f = _feat_stream(feat2d)
    out_f = out_f.reshape(_B, _H, _W, _C).transpose(0, 3, 1, 2)
    return (out_f, out_l)


# speedup vs baseline: 3.8320x; 1.1444x over previous
import functools

import jax
import jax.numpy as jnp
from jax import lax
from jax.experimental import pallas as pl
from jax.experimental.pallas import tpu as pltpu
from jax.experimental.pallas import tpu_sc as plsc

MEMORY_SIZE = 512
CONTRAST_SIZE = 64

_NUM_CORES = 2

_B = 16
_C = 128
_H = 32
_W = 32
_FROWS = _B * _H * _W
_BLK = 8192


_SPB = _B // _NUM_CORES


@functools.partial(
    pl.kernel,
    mesh=plsc.ScalarSubcoreMesh(axis_name="c", num_cores=_NUM_CORES),
    out_type=jax.ShapeDtypeStruct((_B, 1, _H, _W), jnp.float32),
    scratch_types=[
        pltpu.VMEM_SHARED((_SPB, 1, _H, _W), jnp.float32),
        pltpu.SemaphoreType.DMA,
    ],
    compiler_params=pltpu.CompilerParams(
        use_tc_tiling_on_sc=True, skip_device_barrier=True),
)
def _mask_gather(lab_hbm, out_l_hbm, lbuf, lsem):
    cid = lax.axis_index("c")
    base = cid * _SPB

    rd = pltpu.make_async_copy(lab_hbm.at[pl.ds(base, _SPB)], lbuf, lsem)
    rd.start()
    rd.wait()
    wr = pltpu.make_async_copy(lbuf, out_l_hbm.at[pl.ds(base, _SPB)], lsem)
    wr.start()
    wr.wait()


def _feat_body(src_ref, dst_ref):
    dst_ref[...] = src_ref[...]


_feat_stream = pl.pallas_call(
    _feat_body,
    grid=(_FROWS // _BLK,),
    in_specs=[pl.BlockSpec((_BLK, _C), lambda i: (i, 0))],
    out_specs=pl.BlockSpec((_BLK, _C), lambda i: (i, 0)),
    out_shape=jax.ShapeDtypeStruct((_FROWS, _C), jnp.float32),
    compiler_params=pltpu.CompilerParams(skip_device_barrier=True),
)


def kernel(t_feats, labels, teacher_feature_queue, teacher_mask_queue):
    del teacher_feature_queue, teacher_mask_queue
    keys = jax.lax.stop_gradient(t_feats)
    labs = jax.lax.stop_gradient(labels.astype(jnp.float32))
    feat2d = keys.transpose(0, 2, 3, 1).reshape(_FROWS, _C)
    out_l = _mask_gather(labs)
    out_f = _feat_stream(feat2d)
    out_f = out_f.reshape(_B, _H, _W, _C).transpose(0, 3, 1, 2)
    return (out_f, out_l)
